# baseline (speedup 1.0000x reference)
import jax
import jax.numpy as jnp
import numpy as np
from jax.experimental import pallas as pl
from jax.experimental.pallas import tpu as pltpu

_DIA = [100] * 8
_D = len(_DIA)
_T = _DIA[0]
_P = _D * _T
_F = 512
_GCN_DEG = float(_T + 1)

_G_NP = np.zeros((_P, _D), np.float32)
_G_NP[np.arange(_P), np.arange(_P) // _T] = 1.0
_GT_H_NP = np.ascontiguousarray(_G_NP.T) / _T
_G_GCN_NP = _G_NP * (_T / _GCN_DEG)
_MASK2_NP = np.ones((_P, 2), np.float32)
_MASK2_NP[:, 0] = np.arange(_P) % _T <= _T - 4


def _fused_kernel(l_ref, a_ref, v_ref, q0_ref, q1_ref, emb_ref,
                  w1_ref, b1_ref,
                  th1_ref, th2_ref, th3_ref,
                  gw1_ref, gw2_ref, gw3_ref, gw4_ref,
                  ha1_ref, ha2_ref,
                  pack_ref,
                  g_ref, gth_ref, ggcn_ref, mask2_ref,
                  out_ref, hyp_s, gcn_s):
    f32 = jnp.float32
    i = pl.program_id(0)

    @pl.when(i == 0)
    def _step0():
        _compute(l_ref, a_ref, v_ref, q0_ref, q1_ref, emb_ref,
                 w1_ref, b1_ref, th1_ref, th2_ref, th3_ref,
                 gw1_ref, gw2_ref, gw3_ref, gw4_ref,
                 ha1_ref, ha2_ref, pack_ref,
                 g_ref, gth_ref, ggcn_ref, mask2_ref,
                 out_ref, hyp_s, gcn_s)

    srcs = [None,
            lambda: hyp_s[0:_P, :], lambda: gcn_s[0:_P, :],
            lambda: a_ref[:],
            lambda: hyp_s[_P:2 * _P, :], lambda: gcn_s[_P:2 * _P, :],
            lambda: v_ref[:],
            lambda: hyp_s[2 * _P:3 * _P, :], lambda: gcn_s[2 * _P:3 * _P, :]]
    for step in range(1, 9):
        @pl.when(i == step)
        def _copy(read=srcs[step]):
            out_ref[:] = read()


def _compute(l_ref, a_ref, v_ref, q0_ref, q1_ref, emb_ref,
             w1_ref, b1_ref, th1_ref, th2_ref, th3_ref,
             gw1_ref, gw2_ref, gw3_ref, gw4_ref,
             ha1_ref, ha2_ref, pack_ref,
             g_ref, gth_ref, ggcn_ref, mask2_ref,
             out_ref, hyp_s, gcn_s):
    f32 = jnp.float32

    sel = (q1_ref[:] > q0_ref[:]).astype(f32)
    emb = emb_ref[:]
    l_in = (l_ref[:] + emb[0:1, :]) + sel * (emb[1:2, :] - emb[0:1, :])
    feats = (l_in, a_ref[:], v_ref[:])

    w1 = w1_ref[:]
    b1 = b1_ref[:]
    x1 = tuple(jnp.dot(f, w1, preferred_element_type=f32) + b1 for f in feats)

    g = g_ref[:]
    gth = gth_ref[:]
    ggcn = ggcn_ref[:]
    mask = mask2_ref[:][:, 0:1]
    pack = pack_ref[:]
    ewm_c = pack[:, 0:1]
    ewt3_c = pack[:, 1:2]
    wm_c = pack[:, 2:3]
    wt_c = pack[:, 3:4]
    den = wm_c + wt_c
    rden = 1.0 / jnp.where(den == 0.0, 1.0, den)
    wmr = wm_c * rden
    wtr = wt_c * rden
    g_h = tuple(wmr[m * _P:(m + 1) * _P, :] * g for m in range(3))

    hyp = x1
    for th_ref in (th1_ref, th2_ref, th3_ref):
        th = th_ref[:]
        ea1 = jnp.dot(ha1_ref[:], th, preferred_element_type=f32)
        ea2 = jnp.dot(ha2_ref[:], th, preferred_element_type=f32)
        h = tuple(jnp.dot(x, th, preferred_element_type=f32) for x in hyp)
        ef_tri = (h[0] * ewt3_c[0:_P, :]
                  + h[1] * ewt3_c[_P:2 * _P, :]
                  + h[2] * ewt3_c[2 * _P:3 * _P, :]
                  + (ea2 + mask * (ea1 - ea2)))
        new = []
        for m in range(3):
            lo = m * _P
            hw = h[m] * ewm_c[lo:lo + _P, :]
            s = jnp.dot(gth, hw, preferred_element_type=f32)
            mod_t = jnp.dot(g_h[m], s + ea1, preferred_element_type=f32)
            new.append(mod_t + wtr[lo:lo + _P, :] * ef_tri)
        hyp = tuple(new)

    inv_deg = 1.0 / _GCN_DEG
    c2 = 2.0 / _GCN_DEG
    gcn = x1
    for gw_ref in (gw1_ref, gw2_ref, gw3_ref, gw4_ref):
        gw = gw_ref[:]
        h = tuple(jnp.dot(x, gw, preferred_element_type=f32) for x in gcn)
        trid = inv_deg * (h[0] + h[1] + h[2])
        new = []
        for m in range(3):
            s = jnp.dot(gth, h[m], preferred_element_type=f32)
            mod_b = jnp.dot(ggcn, s, preferred_element_type=f32)
            new.append(gcn[m] + mod_b + (trid - c2 * h[m]))
        gcn = tuple(new)

    out_ref[:] = feats[0]
    for m in range(3):
        hyp_s[m * _P:(m + 1) * _P, :] = hyp[m]
        gcn_s[m * _P:(m + 1) * _P, :] = gcn[m]


def kernel(a, v, l, qmask, dia_len, epoch, speaker_emb, fc1_W, fc1_b,
           hyperedge_weight, EW_weight, h_attr1, h_attr2,
           theta1, theta2, theta3, gW1, gW2, gW3, gW4):
    f32 = jnp.float32
    qc = jnp.transpose(qmask, (1, 0, 2)).reshape(_P, 2)
    q0 = qc[:, 0:1]
    q1 = qc[:, 1:2]

    ew = EW_weight[:2 * 3 * _P].reshape(_D, 2, 3 * _T)
    ewm = ew[:, 0, :].reshape(_D, 3, _T).transpose(1, 0, 2).reshape(3 * _P, 1)
    ewt3 = (ew[:, 1, :].reshape(_D, _T, 3).transpose(2, 0, 1)
            .reshape(3 * _P, 1)) * (1.0 / 3.0)

    hw = hyperedge_weight[:_D * (_T + 3)].reshape(_D, _T + 3)
    wm = jnp.broadcast_to(hw[:, :3].transpose(1, 0)[:, :, None],
                          (3, _D, _T)).reshape(3 * _P, 1)
    wt = jnp.broadcast_to(hw[None, :, 3:], (3, _D, _T)).reshape(3 * _P, 1)
    pack = jnp.concatenate([ewm, ewt3, wm, wt], axis=1).astype(f32)

    const = lambda *blk: pl.BlockSpec(blk, lambda k: (0,) * len(blk))
    out = pl.pallas_call(
        _fused_kernel,
        grid=(9,),
        in_specs=[const(_P, _F), const(_P, _F), const(_P, _F),
                  const(_P, 1), const(_P, 1), const(2, _F),
                  const(_F, _F), const(1, _F),
                  const(_F, _F), const(_F, _F), const(_F, _F),
                  const(_F, _F), const(_F, _F), const(_F, _F), const(_F, _F),
                  const(1, _F), const(1, _F), const(3 * _P, 4),
                  const(_P, _D), const(_D, _P), const(_P, _D), const(_P, 2)],
        out_specs=pl.BlockSpec((_P, _F), lambda k: (0, k)),
        out_shape=jax.ShapeDtypeStruct((_P, 9 * _F), f32),
        scratch_shapes=[pltpu.VMEM((3 * _P, _F), f32)] * 2,
        compiler_params=pltpu.CompilerParams(
            dimension_semantics=("arbitrary",)),
    )(l.astype(f32), a.astype(f32), v.astype(f32),
      q0.astype(f32), q1.astype(f32), speaker_emb.astype(f32),
      fc1_W.astype(f32), fc1_b.reshape(1, _F).astype(f32),
      theta1.astype(f32), theta2.astype(f32), theta3.astype(f32),
      gW1.astype(f32), gW2.astype(f32), gW3.astype(f32), gW4.astype(f32),
      h_attr1.reshape(1, _F).astype(f32), h_attr2.reshape(1, _F).astype(f32),
      pack,
      jnp.asarray(_G_NP), jnp.asarray(_GT_H_NP), jnp.asarray(_G_GCN_NP),
      jnp.asarray(_MASK2_NP))
    return out


# speedup vs baseline: 1.0368x; 1.0368x over previous
import jax
import jax.numpy as jnp
import numpy as np
from jax.experimental import pallas as pl

_DIA = [100] * 8
_D = len(_DIA)
_T = _DIA[0]
_P = _D * _T
_F = 512
_GCN_DEG = float(_T + 1)

_G_NP = np.zeros((_P, _D), np.float32)
_G_NP[np.arange(_P), np.arange(_P) // _T] = 1.0
_GT_H_NP = np.ascontiguousarray(_G_NP.T) / _T
_G_GCN_NP = _G_NP * (_T / _GCN_DEG)
_MASK2_NP = np.ones((_P, 2), np.float32)
_MASK2_NP[:, 0] = np.arange(_P) % _T <= _T - 4


def _fused_kernel(l_ref, a_ref, v_ref, q0_ref, q1_ref, emb_ref,
                  w1_ref, b1_ref,
                  th1_ref, th2_ref, th3_ref,
                  gw1_ref, gw2_ref, gw3_ref, gw4_ref,
                  ha1_ref, ha2_ref,
                  pack_ref,
                  g_ref, gth_ref, ggcn_ref, mask2_ref,
                  out_ref):
    f32 = jnp.float32

    sel = (q1_ref[:] > q0_ref[:]).astype(f32)
    emb = emb_ref[:]
    l_in = (l_ref[:] + emb[0:1, :]) + sel * (emb[1:2, :] - emb[0:1, :])
    feats = (l_in, a_ref[:], v_ref[:])

    w1 = w1_ref[:]
    b1 = b1_ref[:]
    x1 = tuple(jnp.dot(f, w1, preferred_element_type=f32) + b1 for f in feats)

    g = g_ref[:]
    gth = gth_ref[:]
    ggcn = ggcn_ref[:]
    mask = mask2_ref[:][:, 0:1]
    pack = pack_ref[:]
    ewm_c = pack[:, 0:1]
    ewt3_c = pack[:, 1:2]
    wm_c = pack[:, 2:3]
    wt_c = pack[:, 3:4]
    den = wm_c + wt_c
    rden = 1.0 / jnp.where(den == 0.0, 1.0, den)
    wmr = wm_c * rden
    wtr = wt_c * rden
    g_h = tuple(wmr[m * _P:(m + 1) * _P, :] * g for m in range(3))

    hyp = x1
    for th_ref in (th1_ref, th2_ref, th3_ref):
        th = th_ref[:]
        ea1 = jnp.dot(ha1_ref[:], th, preferred_element_type=f32)
        ea2 = jnp.dot(ha2_ref[:], th, preferred_element_type=f32)
        h = tuple(jnp.dot(x, th, preferred_element_type=f32) for x in hyp)
        ef_tri = (h[0] * ewt3_c[0:_P, :]
                  + h[1] * ewt3_c[_P:2 * _P, :]
                  + h[2] * ewt3_c[2 * _P:3 * _P, :]
                  + (ea2 + mask * (ea1 - ea2)))
        new = []
        for m in range(3):
            lo = m * _P
            hw = h[m] * ewm_c[lo:lo + _P, :]
            s = jnp.dot(gth, hw, preferred_element_type=f32)
            mod_t = jnp.dot(g_h[m], s + ea1, preferred_element_type=f32)
            new.append(mod_t + wtr[lo:lo + _P, :] * ef_tri)
        hyp = tuple(new)

    inv_deg = 1.0 / _GCN_DEG
    c2 = 2.0 / _GCN_DEG
    gcn = x1
    for gw_ref in (gw1_ref, gw2_ref, gw3_ref, gw4_ref):
        gw = gw_ref[:]
        h = tuple(jnp.dot(x, gw, preferred_element_type=f32) for x in gcn)
        trid = inv_deg * (h[0] + h[1] + h[2])
        new = []
        for m in range(3):
            s = jnp.dot(gth, h[m], preferred_element_type=f32)
            mod_b = jnp.dot(ggcn, s, preferred_element_type=f32)
            new.append(gcn[m] + mod_b + (trid - c2 * h[m]))
        gcn = tuple(new)

    for m in range(3):
        base = 3 * m * _F
        out_ref[:, base:base + _F] = feats[m]
        out_ref[:, base + _F:base + 2 * _F] = hyp[m]
        out_ref[:, base + 2 * _F:base + 3 * _F] = gcn[m]


def kernel(a, v, l, qmask, dia_len, epoch, speaker_emb, fc1_W, fc1_b,
           hyperedge_weight, EW_weight, h_attr1, h_attr2,
           theta1, theta2, theta3, gW1, gW2, gW3, gW4):
    f32 = jnp.float32
    qc = jnp.transpose(qmask, (1, 0, 2)).reshape(_P, 2)
    q0 = qc[:, 0:1]
    q1 = qc[:, 1:2]

    ew = EW_weight[:2 * 3 * _P].reshape(_D, 2, 3 * _T)
    ewm = ew[:, 0, :].reshape(_D, 3, _T).transpose(1, 0, 2).reshape(3 * _P, 1)
    ewt3 = (ew[:, 1, :].reshape(_D, _T, 3).transpose(2, 0, 1)
            .reshape(3 * _P, 1)) * (1.0 / 3.0)

    hw = hyperedge_weight[:_D * (_T + 3)].reshape(_D, _T + 3)
    wm = jnp.broadcast_to(hw[:, :3].transpose(1, 0)[:, :, None],
                          (3, _D, _T)).reshape(3 * _P, 1)
    wt = jnp.broadcast_to(hw[None, :, 3:], (3, _D, _T)).reshape(3 * _P, 1)
    pack = jnp.concatenate([ewm, ewt3, wm, wt], axis=1).astype(f32)

    out = pl.pallas_call(
        _fused_kernel,
        out_shape=jax.ShapeDtypeStruct((_P, 9 * _F), f32),
    )(l.astype(f32), a.astype(f32), v.astype(f32),
      q0.astype(f32), q1.astype(f32), speaker_emb.astype(f32),
      fc1_W.astype(f32), fc1_b.reshape(1, _F).astype(f32),
      theta1.astype(f32), theta2.astype(f32), theta3.astype(f32),
      gW1.astype(f32), gW2.astype(f32), gW3.astype(f32), gW4.astype(f32),
      h_attr1.reshape(1, _F).astype(f32), h_attr2.reshape(1, _F).astype(f32),
      pack,
      jnp.asarray(_G_NP), jnp.asarray(_GT_H_NP), jnp.asarray(_G_GCN_NP),
      jnp.asarray(_MASK2_NP))
    return out
